# baseline (speedup 1.0000x reference)
import jax
import jax.numpy as jnp
from jax import lax
from jax.experimental import pallas as pl
from jax.experimental.pallas import tpu as pltpu
from jax.experimental.pallas import tpu_sc as plsc

N_POINTS = 524288
RES = 16
VPD = 17
N_TAB = VPD * VPD * VPD
F_TAB = 11
TAB_PAD = N_TAB * F_TAB + 5
F_OUT_G = 4
F_OUT_V = 7

NC = 2
NS = 16
NW = NC * NS
PTS_PER_W = N_POINTS // NW
CHUNK = 2048
N_CHUNKS = PTS_PER_W // CHUNK
N_OUTER = N_CHUNKS // 2
GROUPS = CHUNK // 16


def _rsqrt(x):
    i = lax.bitcast_convert_type(x, jnp.int32)
    i = jnp.int32(0x5F3759DF) - lax.shift_right_logical(i, 1)
    y = lax.bitcast_convert_type(i, jnp.float32)
    for _ in range(3):
        y = y * (1.5 - 0.5 * x * y * y)
    return y


def _sc_body(aux_hbm, go_hbm, vo_hbm, tab_v,
             xb0, xb1, yb0, yb1, zb0, zb1, gob0, gob1, vob0, vob1,
             in_sem0, in_sem1, og_sem0, og_sem1, ov_sem0, ov_sem1):
    wid = lax.axis_index("s") * NC + lax.axis_index("c")
    pltpu.sync_copy(aux_hbm.at[pl.ds(3 * N_POINTS, TAB_PAD)], tab_v)
    lanes = lax.iota(jnp.int32, 16)
    base_w = wid * PTS_PER_W

    xbs = (xb0, xb1)
    ybs = (yb0, yb1)
    zbs = (zb0, zb1)
    gobs = (gob0, gob1)
    vobs = (vob0, vob1)
    in_sems = (in_sem0, in_sem1)
    og_sems = (og_sem0, og_sem1)
    ov_sems = (ov_sem0, ov_sem1)

    def issue_in(b, base):
        pltpu.async_copy(aux_hbm.at[pl.ds(base, CHUNK)], xbs[b], in_sems[b])
        pltpu.async_copy(aux_hbm.at[pl.ds(N_POINTS + base, CHUNK)], ybs[b],
                         in_sems[b])
        pltpu.async_copy(aux_hbm.at[pl.ds(2 * N_POINTS + base, CHUNK)],
                         zbs[b], in_sems[b])

    def wait_in(b, base):
        for buf in (xbs[b], ybs[b], zbs[b]):
            pltpu.make_async_copy(aux_hbm.at[pl.ds(base, CHUNK)], buf,
                                  in_sems[b]).wait()

    for b in (0, 1):
        issue_in(b, base_w + b * CHUNK)

    def group_body_for(xb, yb, zb, gob, vob):
        def group_body(gi, c2):
            s = gi * 16
            rows = s + lanes
            x = xb[pl.ds(s, 16)]
            y = yb[pl.ds(s, 16)]
            z = zb[pl.ds(s, 16)]
            px = x * jnp.float32(RES)
            py = y * jnp.float32(RES)
            pz = z * jnp.float32(RES)
            p0x = px.astype(jnp.int32)
            p0y = py.astype(jnp.int32)
            p0z = pz.astype(jnp.int32)
            fx = px - p0x.astype(jnp.float32)
            fy = py - p0y.astype(jnp.float32)
            fz = pz - p0z.astype(jnp.float32)
            zero = jnp.int32(0)
            hi = jnp.int32(RES)
            cx = (jnp.minimum(jnp.maximum(p0x, zero), hi),
                  jnp.minimum(p0x + 1, hi))
            cyo = (jnp.minimum(jnp.maximum(p0y, zero), hi) * VPD,
                   jnp.minimum(p0y + 1, hi) * VPD)
            czo = (jnp.minimum(jnp.maximum(p0z, zero), hi) * (VPD * VPD),
                   jnp.minimum(p0z + 1, hi) * (VPD * VPD))
            wx = (1.0 - fx, fx)
            wy = (1.0 - fy, fy)
            wz = (1.0 - fz, fz)

            acc = [jnp.zeros((16,), jnp.float32) for _ in range(F_TAB)]
            for dx in (0, 1):
                for dy in (0, 1):
                    wxy = wx[dx] * wy[dy]
                    cxy = cx[dx] + cyo[dy]
                    for dz in (0, 1):
                        w = wxy * wz[dz]
                        fidx = (cxy + czo[dz]) * F_TAB
                        for f in range(F_TAB):
                            t = plsc.load_gather(tab_v, [fidx + f])
                            acc[f] = acc[f] + w * t

            g0 = acc[0] * 50.0 + 0.5
            g1 = acc[1] * 50.0 + 0.5
            g2 = acc[2] * 50.0 + 0.5
            g3 = jnp.maximum(acc[3], 0.001)
            sharp = jnp.minimum(jnp.maximum(acc[4], 0.1), 1.0)
            a0, a1, a2 = acc[5], acc[6], acc[7]
            ss = jnp.maximum(a0 * a0 + a1 * a1 + a2 * a2, 1e-30)
            nrm = ss * _rsqrt(ss)
            den = jnp.maximum(nrm, 1e-6)
            ax0 = a0 / den
            ax1 = a1 / den
            ax2 = a2 / den
            am0 = 1.0 / (1.0 + jnp.exp(-acc[8]))
            am1 = 1.0 / (1.0 + jnp.exp(-acc[9]))
            am2 = 1.0 / (1.0 + jnp.exp(-acc[10]))

            gb = rows * F_OUT_G
            for f, val in enumerate((g0, g1, g2, g3)):
                plsc.store_scatter(gob, [gb + f], val)
            vb = rows * F_OUT_V
            for f, val in enumerate((sharp, ax0, ax1, ax2, am0, am1, am2)):
                plsc.store_scatter(vob, [vb + f], val)
            return c2
        return group_body

    def outer(ci2, carry):
        for b in (0, 1):
            ci = ci2 * 2 + b
            base = base_w + ci * CHUNK
            wait_in(b, base)

            @pl.when(ci2 > 0)
            def _wait_out():
                pb = base - 2 * CHUNK
                pltpu.make_async_copy(
                    gobs[b], go_hbm.at[pl.ds(pb * F_OUT_G, CHUNK * F_OUT_G)],
                    og_sems[b]).wait()
                pltpu.make_async_copy(
                    vobs[b], vo_hbm.at[pl.ds(pb * F_OUT_V, CHUNK * F_OUT_V)],
                    ov_sems[b]).wait()

            lax.fori_loop(
                0, GROUPS,
                group_body_for(xbs[b], ybs[b], zbs[b], gobs[b], vobs[b]), 0)

            pltpu.async_copy(
                gobs[b], go_hbm.at[pl.ds(base * F_OUT_G, CHUNK * F_OUT_G)],
                og_sems[b])
            pltpu.async_copy(
                vobs[b], vo_hbm.at[pl.ds(base * F_OUT_V, CHUNK * F_OUT_V)],
                ov_sems[b])

            @pl.when(ci2 < N_OUTER - 1)
            def _next_in():
                issue_in(b, base + 2 * CHUNK)
        return carry

    lax.fori_loop(0, N_OUTER, outer, 0)

    for b in (0, 1):
        lb = base_w + ((N_OUTER - 1) * 2 + b) * CHUNK
        pltpu.make_async_copy(
            gobs[b], go_hbm.at[pl.ds(lb * F_OUT_G, CHUNK * F_OUT_G)],
            og_sems[b]).wait()
        pltpu.make_async_copy(
            vobs[b], vo_hbm.at[pl.ds(lb * F_OUT_V, CHUNK * F_OUT_V)],
            ov_sems[b]).wait()


@jax.jit
def _run(aux):
    mesh = plsc.VectorSubcoreMesh(core_axis_name="c", subcore_axis_name="s")
    f = pl.kernel(
        _sc_body,
        out_type=(
            jax.ShapeDtypeStruct((N_POINTS * F_OUT_G,), jnp.float32),
            jax.ShapeDtypeStruct((N_POINTS * F_OUT_V,), jnp.float32),
        ),
        mesh=mesh,
        compiler_params=pltpu.CompilerParams(needs_layout_passes=False),
        scratch_types=[
            pltpu.VMEM((TAB_PAD,), jnp.float32),
            pltpu.VMEM((CHUNK,), jnp.float32),
            pltpu.VMEM((CHUNK,), jnp.float32),
            pltpu.VMEM((CHUNK,), jnp.float32),
            pltpu.VMEM((CHUNK,), jnp.float32),
            pltpu.VMEM((CHUNK,), jnp.float32),
            pltpu.VMEM((CHUNK,), jnp.float32),
            pltpu.VMEM((CHUNK * F_OUT_G,), jnp.float32),
            pltpu.VMEM((CHUNK * F_OUT_G,), jnp.float32),
            pltpu.VMEM((CHUNK * F_OUT_V,), jnp.float32),
            pltpu.VMEM((CHUNK * F_OUT_V,), jnp.float32),
            pltpu.SemaphoreType.DMA,
            pltpu.SemaphoreType.DMA,
            pltpu.SemaphoreType.DMA,
            pltpu.SemaphoreType.DMA,
            pltpu.SemaphoreType.DMA,
            pltpu.SemaphoreType.DMA,
        ],
    )
    return f(aux)


def kernel(input, gaussian_table, vmf_table):
    tab = jnp.concatenate(
        [gaussian_table[:N_TAB, :F_OUT_G], vmf_table[:N_TAB, :F_OUT_V]],
        axis=1).reshape(-1)
    aux = jnp.concatenate(
        [input[:, 0], input[:, 1], input[:, 2], tab,
         jnp.zeros((5,), jnp.float32)])
    go, vo = _run(aux)
    return (go.reshape(N_POINTS, F_OUT_G), vo.reshape(N_POINTS, F_OUT_V))


# speedup vs baseline: 1.0503x; 1.0503x over previous
import jax
import jax.numpy as jnp
from jax import lax
from jax.experimental import pallas as pl
from jax.experimental.pallas import tpu as pltpu
from jax.experimental.pallas import tpu_sc as plsc

N_POINTS = 524288
RES = 16
VPD = 17
N_TAB = VPD * VPD * VPD
F_TAB = 11
F_OUT_G = 4
F_OUT_V = 7

NC = 2
NS = 16
NW = NC * NS
PTS_PER_W = N_POINTS // NW
CHUNK = 2048
N_CHUNKS = PTS_PER_W // CHUNK
N_OUTER = N_CHUNKS // 2
GROUPS = CHUNK // 16


def _rsqrt(x):
    i = lax.bitcast_convert_type(x, jnp.int32)
    i = jnp.int32(0x5F3759DF) - lax.shift_right_logical(i, 1)
    y = lax.bitcast_convert_type(i, jnp.float32)
    for _ in range(3):
        y = y * (1.5 - 0.5 * x * y * y)
    return y


def _sc_body(x_hbm, y_hbm, z_hbm, tab_hbm, go_hbm, vo_hbm, tab_v,
             xb0, xb1, yb0, yb1, zb0, zb1, gob0, gob1, vob0, vob1,
             in_sem0, in_sem1, og_sem0, og_sem1, ov_sem0, ov_sem1):
    wid = lax.axis_index("s") * NC + lax.axis_index("c")
    pltpu.sync_copy(tab_hbm, tab_v)
    lanes = lax.iota(jnp.int32, 16)
    base_w = wid * PTS_PER_W

    xbs = (xb0, xb1)
    ybs = (yb0, yb1)
    zbs = (zb0, zb1)
    gobs = (gob0, gob1)
    vobs = (vob0, vob1)
    in_sems = (in_sem0, in_sem1)
    og_sems = (og_sem0, og_sem1)
    ov_sems = (ov_sem0, ov_sem1)

    def issue_in(b, base):
        pltpu.async_copy(x_hbm.at[pl.ds(base, CHUNK)], xbs[b], in_sems[b])
        pltpu.async_copy(y_hbm.at[pl.ds(base, CHUNK)], ybs[b], in_sems[b])
        pltpu.async_copy(z_hbm.at[pl.ds(base, CHUNK)], zbs[b], in_sems[b])

    def wait_in(b, base):
        pltpu.make_async_copy(x_hbm.at[pl.ds(base, CHUNK)], xbs[b],
                              in_sems[b]).wait()
        pltpu.make_async_copy(y_hbm.at[pl.ds(base, CHUNK)], ybs[b],
                              in_sems[b]).wait()
        pltpu.make_async_copy(z_hbm.at[pl.ds(base, CHUNK)], zbs[b],
                              in_sems[b]).wait()

    for b in (0, 1):
        issue_in(b, base_w + b * CHUNK)

    def group_body_for(xb, yb, zb, gob, vob):
        def group_body(gi, c2):
            s = gi * 16
            rows = s + lanes
            x = xb[pl.ds(s, 16)]
            y = yb[pl.ds(s, 16)]
            z = zb[pl.ds(s, 16)]
            px = x * jnp.float32(RES)
            py = y * jnp.float32(RES)
            pz = z * jnp.float32(RES)
            p0x = px.astype(jnp.int32)
            p0y = py.astype(jnp.int32)
            p0z = pz.astype(jnp.int32)
            fx = px - p0x.astype(jnp.float32)
            fy = py - p0y.astype(jnp.float32)
            fz = pz - p0z.astype(jnp.float32)
            zero = jnp.int32(0)
            hi = jnp.int32(RES)
            cx = (jnp.minimum(jnp.maximum(p0x, zero), hi),
                  jnp.minimum(p0x + 1, hi))
            cyo = (jnp.minimum(jnp.maximum(p0y, zero), hi) * VPD,
                   jnp.minimum(p0y + 1, hi) * VPD)
            czo = (jnp.minimum(jnp.maximum(p0z, zero), hi) * (VPD * VPD),
                   jnp.minimum(p0z + 1, hi) * (VPD * VPD))
            wx = (1.0 - fx, fx)
            wy = (1.0 - fy, fy)
            wz = (1.0 - fz, fz)

            acc = [jnp.zeros((16,), jnp.float32) for _ in range(F_TAB)]
            for dx in (0, 1):
                for dy in (0, 1):
                    wxy = wx[dx] * wy[dy]
                    cxy = cx[dx] + cyo[dy]
                    for dz in (0, 1):
                        w = wxy * wz[dz]
                        fidx = (cxy + czo[dz]) * F_TAB
                        for f in range(F_TAB):
                            t = plsc.load_gather(tab_v, [fidx + f])
                            acc[f] = acc[f] + w * t

            g0 = acc[0] * 50.0 + 0.5
            g1 = acc[1] * 50.0 + 0.5
            g2 = acc[2] * 50.0 + 0.5
            g3 = jnp.maximum(acc[3], 0.001)
            sharp = jnp.minimum(jnp.maximum(acc[4], 0.1), 1.0)
            a0, a1, a2 = acc[5], acc[6], acc[7]
            ss = jnp.maximum(a0 * a0 + a1 * a1 + a2 * a2, 1e-30)
            nrm = ss * _rsqrt(ss)
            den = jnp.maximum(nrm, 1e-6)
            ax0 = a0 / den
            ax1 = a1 / den
            ax2 = a2 / den
            am0 = 1.0 / (1.0 + jnp.exp(-acc[8]))
            am1 = 1.0 / (1.0 + jnp.exp(-acc[9]))
            am2 = 1.0 / (1.0 + jnp.exp(-acc[10]))

            gb = rows * F_OUT_G
            for f, val in enumerate((g0, g1, g2, g3)):
                plsc.store_scatter(gob, [gb + f], val)
            vb = rows * F_OUT_V
            for f, val in enumerate((sharp, ax0, ax1, ax2, am0, am1, am2)):
                plsc.store_scatter(vob, [vb + f], val)
            return c2
        return group_body

    def outer(ci2, carry):
        for b in (0, 1):
            ci = ci2 * 2 + b
            base = base_w + ci * CHUNK
            wait_in(b, base)

            @pl.when(ci2 > 0)
            def _wait_out():
                pb = base - 2 * CHUNK
                pltpu.make_async_copy(
                    gobs[b], go_hbm.at[pl.ds(pb * F_OUT_G, CHUNK * F_OUT_G)],
                    og_sems[b]).wait()
                pltpu.make_async_copy(
                    vobs[b], vo_hbm.at[pl.ds(pb * F_OUT_V, CHUNK * F_OUT_V)],
                    ov_sems[b]).wait()

            lax.fori_loop(
                0, GROUPS,
                group_body_for(xbs[b], ybs[b], zbs[b], gobs[b], vobs[b]), 0)

            pltpu.async_copy(
                gobs[b], go_hbm.at[pl.ds(base * F_OUT_G, CHUNK * F_OUT_G)],
                og_sems[b])
            pltpu.async_copy(
                vobs[b], vo_hbm.at[pl.ds(base * F_OUT_V, CHUNK * F_OUT_V)],
                ov_sems[b])

            @pl.when(ci2 < N_OUTER - 1)
            def _next_in():
                issue_in(b, base + 2 * CHUNK)
        return carry

    lax.fori_loop(0, N_OUTER, outer, 0)

    for b in (0, 1):
        lb = base_w + ((N_OUTER - 1) * 2 + b) * CHUNK
        pltpu.make_async_copy(
            gobs[b], go_hbm.at[pl.ds(lb * F_OUT_G, CHUNK * F_OUT_G)],
            og_sems[b]).wait()
        pltpu.make_async_copy(
            vobs[b], vo_hbm.at[pl.ds(lb * F_OUT_V, CHUNK * F_OUT_V)],
            ov_sems[b]).wait()


@jax.jit
def _run(x, y, z, tab):
    mesh = plsc.VectorSubcoreMesh(core_axis_name="c", subcore_axis_name="s")
    f = pl.kernel(
        _sc_body,
        out_type=(
            jax.ShapeDtypeStruct((N_POINTS * F_OUT_G,), jnp.float32),
            jax.ShapeDtypeStruct((N_POINTS * F_OUT_V,), jnp.float32),
        ),
        mesh=mesh,
        compiler_params=pltpu.CompilerParams(needs_layout_passes=False),
        scratch_types=[
            pltpu.VMEM((N_TAB * F_TAB,), jnp.float32),
            pltpu.VMEM((CHUNK,), jnp.float32),
            pltpu.VMEM((CHUNK,), jnp.float32),
            pltpu.VMEM((CHUNK,), jnp.float32),
            pltpu.VMEM((CHUNK,), jnp.float32),
            pltpu.VMEM((CHUNK,), jnp.float32),
            pltpu.VMEM((CHUNK,), jnp.float32),
            pltpu.VMEM((CHUNK * F_OUT_G,), jnp.float32),
            pltpu.VMEM((CHUNK * F_OUT_G,), jnp.float32),
            pltpu.VMEM((CHUNK * F_OUT_V,), jnp.float32),
            pltpu.VMEM((CHUNK * F_OUT_V,), jnp.float32),
            pltpu.SemaphoreType.DMA,
            pltpu.SemaphoreType.DMA,
            pltpu.SemaphoreType.DMA,
            pltpu.SemaphoreType.DMA,
            pltpu.SemaphoreType.DMA,
            pltpu.SemaphoreType.DMA,
        ],
    )
    return f(x, y, z, tab)


def kernel(input, gaussian_table, vmf_table):
    x = input[:, 0]
    y = input[:, 1]
    z = input[:, 2]
    tab = jnp.concatenate(
        [gaussian_table[:N_TAB, :F_OUT_G], vmf_table[:N_TAB, :F_OUT_V]],
        axis=1).reshape(-1)
    go, vo = _run(x, y, z, tab)
    return (go.reshape(N_POINTS, F_OUT_G), vo.reshape(N_POINTS, F_OUT_V))
